# baseline (speedup 1.0000x reference)
import math

import numpy as np
import jax
import jax.numpy as jnp
from jax import lax
from jax.experimental import pallas as pl
from jax.experimental.pallas import tpu as pltpu
from jax.experimental.pallas import tpu_sc as plsc

D_DIM = 256
D_HALF = 128
BATCH = 128
SEQ = 1500
SEQ_PAD = 1536
CH = 16
N_STEP = 93
SC_SEQ = CH * N_STEP
TC_TAIL = SEQ - SC_SEQ
N_PAIR = 16
B_PER_P = BATCH // N_PAIR
SCALE = math.sqrt(D_DIM)
LANES = 16
B_TC = 8


def _pe_np():
    position = np.arange(0.0, SEQ, dtype=np.float64)[:, None]
    div_term = np.exp(
        np.arange(0.0, D_DIM, 2, dtype=np.float64) * -(math.log(10000.0) / D_DIM)
    )
    ang = position * div_term
    pe = np.zeros((SEQ_PAD, D_DIM), dtype=np.float32)
    pe[:SEQ, 0::2] = np.sin(ang)
    pe[:SEQ, 1::2] = np.cos(ang)
    return pe


_PE = _pe_np()
_PE_T_CONST = np.ascontiguousarray(
    _PE[:SC_SEQ].reshape(N_STEP, CH, D_DIM).transpose(0, 2, 1)
)
_PE_TAIL_CONST = np.ascontiguousarray(_PE[SC_SEQ:SEQ])


def _sc_body(
    x_hbm, table_hbm, pet_hbm, out_hbm,
    idx_v, table_v, wb0, wb1, pe0, pe1,
    wsem0, wsem1, psem0, psem1,
):
    pair = lax.axis_index("s")
    half = lax.axis_index("c")
    b0 = pair * B_PER_P
    d0 = half * D_HALF
    wb = [wb0, wb1]
    peb = [pe0, pe1]
    wsem = [wsem0, wsem1]
    psem = [psem0, psem1]

    pltpu.sync_copy(x_hbm.at[pair], idx_v)
    pltpu.sync_copy(table_hbm.at[:, pl.ds(d0, D_HALF)], table_v)

    lane = lax.iota(jnp.int32, LANES)

    def prefetch_pe(i, p):
        pltpu.async_copy(pet_hbm.at[i, pl.ds(d0, D_HALF)], peb[p], psem[p])

    def wait_pe(p):
        pltpu.make_async_copy(
            pet_hbm.at[0, pl.ds(0, D_HALF)], peb[p], psem[p]
        ).wait()

    def wait_write(p):
        for b in range(B_PER_P):
            pltpu.make_async_copy(
                wb[p].at[b], out_hbm.at[0, pl.ds(0, CH), pl.ds(0, D_HALF)], wsem[p]
            ).wait()

    def step(i, p, q):
        @pl.when(i >= 2)
        def _():
            wait_write(p)

        @pl.when(i + 1 < N_STEP)
        def _():
            prefetch_pe(i + 1, q)

        idx_vecs = [idx_v[b, pl.ds(i * CH, CH)] for b in range(B_PER_P)]

        wait_pe(p)

        def col_body(j, carry):
            cols = jnp.full((LANES,), j, dtype=jnp.int32)
            pe_col = peb[p][j, :]
            for b in range(B_PER_P):
                g = plsc.load_gather(table_v, [idx_vecs[b], cols])
                val = g * SCALE + pe_col
                plsc.store_scatter(
                    wb[p], [jnp.full((LANES,), b, dtype=jnp.int32), lane, cols], val
                )
            return carry

        lax.fori_loop(0, D_HALF, col_body, 0)

        for b in range(B_PER_P):
            pltpu.async_copy(
                wb[p].at[b],
                out_hbm.at[b0 + b, pl.ds(i * CH, CH), pl.ds(d0, D_HALF)],
                wsem[p],
            )

    prefetch_pe(0, 0)
    step(0, 0, 1)

    def loop_body(it, carry):
        base = 1 + it * 2
        step(base, 1, 0)
        step(base + 1, 0, 1)
        return carry

    lax.fori_loop(0, (N_STEP - 1) // 2, loop_body, 0)

    wait_write(1)
    wait_write(0)


def _tc_tail_body(xt_ref, table_ref, pe_ref, out_in_ref, out_ref, acc_ref, sem):
    del out_in_ref
    i = pl.program_id(0)
    table = table_ref[...]
    pe = pe_ref[...]
    for j in range(B_TC):
        row = xt_ref[j, :]
        oh = (row[:, None] == lax.broadcasted_iota(jnp.int32, (TC_TAIL, D_DIM), 1))
        acc = jax.lax.dot(
            oh.astype(jnp.float32), table, precision=lax.Precision.HIGHEST
        )
        acc_ref[j] = acc * SCALE + pe
    copy = pltpu.make_async_copy(
        acc_ref,
        out_ref.at[pl.ds(i * B_TC, B_TC), pl.ds(SC_SEQ, TC_TAIL)],
        sem,
    )
    copy.start()
    copy.wait()


@jax.jit
def _impl(x, table):
    pet = jnp.asarray(_PE_T_CONST)
    pe_tail = jnp.asarray(_PE_TAIL_CONST)
    mesh = plsc.VectorSubcoreMesh(core_axis_name="c", subcore_axis_name="s")
    k = pl.kernel(
        _sc_body,
        mesh=mesh,
        out_type=jax.ShapeDtypeStruct((BATCH, SEQ, D_DIM), jnp.float32),
        scratch_types=[
            pltpu.VMEM((B_PER_P, SEQ_PAD), jnp.int32),
            pltpu.VMEM((D_DIM, D_HALF), jnp.float32),
            pltpu.VMEM((B_PER_P, CH, D_HALF), jnp.float32),
            pltpu.VMEM((B_PER_P, CH, D_HALF), jnp.float32),
            pltpu.VMEM((D_HALF, CH), jnp.float32),
            pltpu.VMEM((D_HALF, CH), jnp.float32),
            pltpu.SemaphoreType.DMA,
            pltpu.SemaphoreType.DMA,
            pltpu.SemaphoreType.DMA,
            pltpu.SemaphoreType.DMA,
        ],
        compiler_params=pltpu.CompilerParams(needs_layout_passes=False),
    )
    xp = jnp.pad(x, ((0, 0), (0, SEQ_PAD - SEQ))).reshape(N_PAIR, B_PER_P, SEQ_PAD)
    out_sc = k(xp, table, pet)

    xt = lax.slice(x, (0, SC_SEQ), (BATCH, SEQ))
    out = pl.pallas_call(
        _tc_tail_body,
        grid=(BATCH // B_TC,),
        in_specs=[
            pl.BlockSpec((B_TC, TC_TAIL), lambda i: (i, 0)),
            pl.BlockSpec((D_DIM, D_DIM), lambda i: (0, 0)),
            pl.BlockSpec((TC_TAIL, D_DIM), lambda i: (0, 0)),
            pl.BlockSpec(memory_space=pl.ANY),
        ],
        out_specs=pl.BlockSpec(memory_space=pl.ANY),
        out_shape=jax.ShapeDtypeStruct((BATCH, SEQ, D_DIM), jnp.float32),
        scratch_shapes=[
            pltpu.VMEM((B_TC, TC_TAIL, D_DIM), jnp.float32),
            pltpu.SemaphoreType.DMA,
        ],
        input_output_aliases={3: 0},
    )(xt, table, pe_tail, out_sc)
    return out


def kernel(x, table):
    return _impl(x, table)


# speedup vs baseline: 2.6533x; 2.6533x over previous
import math

import numpy as np
import jax
import jax.numpy as jnp
from jax import lax
from jax.experimental import pallas as pl
from jax.experimental.pallas import tpu as pltpu
from jax.experimental.pallas import tpu_sc as plsc

D_DIM = 256
D_HALF = 128
BATCH = 128
SEQ = 1500
SEQ_PAD = 1536
CH = 16
N_STEP = 93
SC_SEQ = CH * N_STEP
TC_TAIL = SEQ - SC_SEQ
N_PAIR = 16
B_PER_P = BATCH // N_PAIR
SCALE = math.sqrt(D_DIM)
LANES = 16
N_G = D_HALF // LANES
B_TC = 8


def _pe_np():
    position = np.arange(0.0, SEQ, dtype=np.float64)[:, None]
    div_term = np.exp(
        np.arange(0.0, D_DIM, 2, dtype=np.float64) * -(math.log(10000.0) / D_DIM)
    )
    ang = position * div_term
    pe = np.zeros((SEQ_PAD, D_DIM), dtype=np.float32)
    pe[:SEQ, 0::2] = np.sin(ang)
    pe[:SEQ, 1::2] = np.cos(ang)
    return pe


_PE = _pe_np()
_PE_T_CONST = np.ascontiguousarray(_PE[:SC_SEQ].reshape(N_STEP, CH, D_DIM))
_PE_TAIL_CONST = np.ascontiguousarray(_PE[SC_SEQ:SEQ])


def _take16(vec, idx):
    dnums = lax.GatherDimensionNumbers(
        offset_dims=(), collapsed_slice_dims=(0,), start_index_map=(0,)
    )
    return lax.gather(
        vec, idx[:, None], dnums, (1,),
        mode=lax.GatherScatterMode.PROMISE_IN_BOUNDS,
    )


def _sc_body(
    x_hbm, table_hbm, pet_hbm, out_hbm,
    idx_v, table_v, wb0, wb1, pe0, pe1,
    wsem0, wsem1, psem0, psem1,
):
    pair = lax.axis_index("s")
    half = lax.axis_index("c")
    b0 = pair * B_PER_P
    d0 = half * D_HALF
    wb = [wb0, wb1]
    peb = [pe0, pe1]
    wsem = [wsem0, wsem1]
    psem = [psem0, psem1]

    pltpu.sync_copy(x_hbm.at[pair], idx_v)
    pltpu.sync_copy(table_hbm.at[:, pl.ds(d0, D_HALF)], table_v)

    lane = lax.iota(jnp.int32, LANES)
    colv = [lane + 16 * g for g in range(N_G)]

    def prefetch_pe(i, p):
        pltpu.async_copy(pet_hbm.at[i, :, pl.ds(d0, D_HALF)], peb[p], psem[p])

    def wait_pe(p):
        pltpu.make_async_copy(
            pet_hbm.at[0, :, pl.ds(0, D_HALF)], peb[p], psem[p]
        ).wait()

    def wait_write(p):
        for b in range(B_PER_P):
            pltpu.make_async_copy(
                wb[p].at[b], out_hbm.at[0, pl.ds(0, CH), pl.ds(0, D_HALF)], wsem[p]
            ).wait()

    def step(i, p, q):
        @pl.when(i >= 2)
        def _():
            wait_write(p)

        @pl.when(i + 1 < N_STEP)
        def _():
            prefetch_pe(i + 1, q)

        idx_vecs = [idx_v[b, pl.ds(i * CH, CH)] for b in range(B_PER_P)]

        wait_pe(p)

        def s_body(s, carry):
            sv = jnp.full((LANES,), s, dtype=jnp.int32)
            pe_vecs = [peb[p][s, pl.ds(16 * g, LANES)] for g in range(N_G)]
            for b in range(B_PER_P):
                spl = _take16(idx_vecs[b], sv)
                for g in range(N_G):
                    gv = plsc.load_gather(table_v, [spl, colv[g]])
                    wb[p][b, s, pl.ds(16 * g, LANES)] = gv * SCALE + pe_vecs[g]
            return carry

        lax.fori_loop(0, CH, s_body, 0)

        for b in range(B_PER_P):
            pltpu.async_copy(
                wb[p].at[b],
                out_hbm.at[b0 + b, pl.ds(i * CH, CH), pl.ds(d0, D_HALF)],
                wsem[p],
            )

    prefetch_pe(0, 0)
    step(0, 0, 1)

    def loop_body(it, carry):
        base = 1 + it * 2
        step(base, 1, 0)
        step(base + 1, 0, 1)
        return carry

    lax.fori_loop(0, (N_STEP - 1) // 2, loop_body, 0)

    wait_write(1)
    wait_write(0)


def _tc_tail_body(xt_ref, table_ref, pe_ref, out_in_ref, out_ref, acc_ref, sem):
    del out_in_ref
    i = pl.program_id(0)
    table = table_ref[...]
    pe = pe_ref[...]
    for j in range(B_TC):
        row = xt_ref[j, :]
        oh = (row[:, None] == lax.broadcasted_iota(jnp.int32, (TC_TAIL, D_DIM), 1))
        acc = jax.lax.dot(
            oh.astype(jnp.float32), table, precision=lax.Precision.HIGHEST
        )
        acc_ref[j] = acc * SCALE + pe
    copy = pltpu.make_async_copy(
        acc_ref,
        out_ref.at[pl.ds(i * B_TC, B_TC), pl.ds(SC_SEQ, TC_TAIL)],
        sem,
    )
    copy.start()
    copy.wait()


@jax.jit
def _impl(x, table):
    pet = jnp.asarray(_PE_T_CONST)
    pe_tail = jnp.asarray(_PE_TAIL_CONST)
    mesh = plsc.VectorSubcoreMesh(core_axis_name="c", subcore_axis_name="s")
    k = pl.kernel(
        _sc_body,
        mesh=mesh,
        out_type=jax.ShapeDtypeStruct((BATCH, SEQ, D_DIM), jnp.float32),
        scratch_types=[
            pltpu.VMEM((B_PER_P, SEQ_PAD), jnp.int32),
            pltpu.VMEM((D_DIM, D_HALF), jnp.float32),
            pltpu.VMEM((B_PER_P, CH, D_HALF), jnp.float32),
            pltpu.VMEM((B_PER_P, CH, D_HALF), jnp.float32),
            pltpu.VMEM((CH, D_HALF), jnp.float32),
            pltpu.VMEM((CH, D_HALF), jnp.float32),
            pltpu.SemaphoreType.DMA,
            pltpu.SemaphoreType.DMA,
            pltpu.SemaphoreType.DMA,
            pltpu.SemaphoreType.DMA,
        ],
        compiler_params=pltpu.CompilerParams(needs_layout_passes=False),
    )
    xp = jnp.pad(x, ((0, 0), (0, SEQ_PAD - SEQ))).reshape(N_PAIR, B_PER_P, SEQ_PAD)
    out_sc = k(xp, table, pet)

    xt = lax.slice(x, (0, SC_SEQ), (BATCH, SEQ))
    out = pl.pallas_call(
        _tc_tail_body,
        grid=(BATCH // B_TC,),
        in_specs=[
            pl.BlockSpec((B_TC, TC_TAIL), lambda i: (i, 0)),
            pl.BlockSpec((D_DIM, D_DIM), lambda i: (0, 0)),
            pl.BlockSpec((TC_TAIL, D_DIM), lambda i: (0, 0)),
            pl.BlockSpec(memory_space=pl.ANY),
        ],
        out_specs=pl.BlockSpec(memory_space=pl.ANY),
        out_shape=jax.ShapeDtypeStruct((BATCH, SEQ, D_DIM), jnp.float32),
        scratch_shapes=[
            pltpu.VMEM((B_TC, TC_TAIL, D_DIM), jnp.float32),
            pltpu.SemaphoreType.DMA,
        ],
        input_output_aliases={3: 0},
    )(xt, table, pe_tail, out_sc)
    return out


def kernel(x, table):
    return _impl(x, table)


# speedup vs baseline: 3.3457x; 1.2610x over previous
import math

import numpy as np
import jax
import jax.numpy as jnp
from jax import lax
from jax.experimental import pallas as pl
from jax.experimental.pallas import tpu as pltpu
from jax.experimental.pallas import tpu_sc as plsc

D_DIM = 256
D_HALF = 128
BATCH = 128
SEQ = 1500
SEQ_PAD = 1536
CH = 16
N_STEP = 93
SC_SEQ = CH * N_STEP
TC_TAIL = SEQ - SC_SEQ
N_PAIR = 16
B_PER_P = BATCH // N_PAIR
SCALE = math.sqrt(D_DIM)
LANES = 16
N_G = D_HALF // LANES
B_TC = 8


def _pe_np():
    position = np.arange(0.0, SEQ, dtype=np.float64)[:, None]
    div_term = np.exp(
        np.arange(0.0, D_DIM, 2, dtype=np.float64) * -(math.log(10000.0) / D_DIM)
    )
    ang = position * div_term
    pe = np.zeros((SEQ_PAD, D_DIM), dtype=np.float32)
    pe[:SEQ, 0::2] = np.sin(ang)
    pe[:SEQ, 1::2] = np.cos(ang)
    return pe


_PE = _pe_np()
_PE_T_CONST = np.ascontiguousarray(_PE[:SC_SEQ].reshape(N_STEP, CH, D_DIM))
_PE_TAIL_CONST = np.ascontiguousarray(_PE[SC_SEQ:SEQ])


def _take16(vec, idx):
    dnums = lax.GatherDimensionNumbers(
        offset_dims=(), collapsed_slice_dims=(0,), start_index_map=(0,)
    )
    return lax.gather(
        vec, idx[:, None], dnums, (1,),
        mode=lax.GatherScatterMode.PROMISE_IN_BOUNDS,
    )


def _sc_body(
    x_hbm, table_hbm, pet_hbm, out_hbm,
    idx_v, table_v, wb0, wb1, pe0, pe1,
    wsem0, wsem1, psem0, psem1,
):
    pair = lax.axis_index("s")
    half = lax.axis_index("c")
    b0 = pair * B_PER_P
    d0 = half * D_HALF
    wb = [wb0, wb1]
    peb = [pe0, pe1]
    wsem = [wsem0, wsem1]
    psem = [psem0, psem1]

    pltpu.sync_copy(x_hbm.at[pair], idx_v)
    pltpu.sync_copy(table_hbm.at[:, pl.ds(d0, D_HALF)], table_v)

    lane = lax.iota(jnp.int32, LANES)
    colv = [lane + 16 * g for g in range(N_G)]

    def prefetch_pe(i, p):
        pltpu.async_copy(pet_hbm.at[i, :, pl.ds(d0, D_HALF)], peb[p], psem[p])

    def wait_pe(p):
        pltpu.make_async_copy(
            pet_hbm.at[0, :, pl.ds(0, D_HALF)], peb[p], psem[p]
        ).wait()

    def wait_write(p):
        for b in range(B_PER_P):
            pltpu.make_async_copy(
                wb[p].at[b], out_hbm.at[0, pl.ds(0, CH), pl.ds(0, D_HALF)], wsem[p]
            ).wait()

    def step(i, p, q):
        @pl.when(i >= 2)
        def _():
            wait_write(p)

        @pl.when(i + 1 < N_STEP)
        def _():
            prefetch_pe(i + 1, q)

        idx_vecs = [idx_v[b, pl.ds(i * CH, CH)] for b in range(B_PER_P)]

        wait_pe(p)

        @plsc.parallel_loop(0, CH, 1)
        def _(s):
            sv = jnp.full((LANES,), s, dtype=jnp.int32)
            pe_vecs = [peb[p][s, pl.ds(16 * g, LANES)] for g in range(N_G)]
            for b in range(B_PER_P):
                spl = _take16(idx_vecs[b], sv)
                for g in range(N_G):
                    gv = plsc.load_gather(table_v, [spl, colv[g]])
                    wb[p][b, s, pl.ds(16 * g, LANES)] = gv * SCALE + pe_vecs[g]

        for b in range(B_PER_P):
            pltpu.async_copy(
                wb[p].at[b],
                out_hbm.at[b0 + b, pl.ds(i * CH, CH), pl.ds(d0, D_HALF)],
                wsem[p],
            )

    prefetch_pe(0, 0)
    step(0, 0, 1)

    def loop_body(it, carry):
        base = 1 + it * 2
        step(base, 1, 0)
        step(base + 1, 0, 1)
        return carry

    lax.fori_loop(0, (N_STEP - 1) // 2, loop_body, 0)

    wait_write(1)
    wait_write(0)


def _tc_tail_body(xt_ref, table_ref, pe_ref, out_in_ref, out_ref, acc_ref, sem):
    del out_in_ref
    i = pl.program_id(0)
    table = table_ref[...]
    pe = pe_ref[...]
    for j in range(B_TC):
        row = xt_ref[j, :]
        oh = (row[:, None] == lax.broadcasted_iota(jnp.int32, (TC_TAIL, D_DIM), 1))
        acc = jax.lax.dot(
            oh.astype(jnp.float32), table, precision=lax.Precision.HIGHEST
        )
        acc_ref[j] = acc * SCALE + pe
    copy = pltpu.make_async_copy(
        acc_ref,
        out_ref.at[pl.ds(i * B_TC, B_TC), pl.ds(SC_SEQ, TC_TAIL)],
        sem,
    )
    copy.start()
    copy.wait()


@jax.jit
def _impl(x, table):
    pet = jnp.asarray(_PE_T_CONST)
    pe_tail = jnp.asarray(_PE_TAIL_CONST)
    mesh = plsc.VectorSubcoreMesh(core_axis_name="c", subcore_axis_name="s")
    k = pl.kernel(
        _sc_body,
        mesh=mesh,
        out_type=jax.ShapeDtypeStruct((BATCH, SEQ, D_DIM), jnp.float32),
        scratch_types=[
            pltpu.VMEM((B_PER_P, SEQ_PAD), jnp.int32),
            pltpu.VMEM((D_DIM, D_HALF), jnp.float32),
            pltpu.VMEM((B_PER_P, CH, D_HALF), jnp.float32),
            pltpu.VMEM((B_PER_P, CH, D_HALF), jnp.float32),
            pltpu.VMEM((CH, D_HALF), jnp.float32),
            pltpu.VMEM((CH, D_HALF), jnp.float32),
            pltpu.SemaphoreType.DMA,
            pltpu.SemaphoreType.DMA,
            pltpu.SemaphoreType.DMA,
            pltpu.SemaphoreType.DMA,
        ],
        compiler_params=pltpu.CompilerParams(needs_layout_passes=False),
    )
    xp = jnp.pad(x, ((0, 0), (0, SEQ_PAD - SEQ))).reshape(N_PAIR, B_PER_P, SEQ_PAD)
    out_sc = k(xp, table, pet)

    xt = lax.slice(x, (0, SC_SEQ), (BATCH, SEQ))
    out = pl.pallas_call(
        _tc_tail_body,
        grid=(BATCH // B_TC,),
        in_specs=[
            pl.BlockSpec((B_TC, TC_TAIL), lambda i: (i, 0)),
            pl.BlockSpec((D_DIM, D_DIM), lambda i: (0, 0)),
            pl.BlockSpec((TC_TAIL, D_DIM), lambda i: (0, 0)),
            pl.BlockSpec(memory_space=pl.ANY),
        ],
        out_specs=pl.BlockSpec(memory_space=pl.ANY),
        out_shape=jax.ShapeDtypeStruct((BATCH, SEQ, D_DIM), jnp.float32),
        scratch_shapes=[
            pltpu.VMEM((B_TC, TC_TAIL, D_DIM), jnp.float32),
            pltpu.SemaphoreType.DMA,
        ],
        input_output_aliases={3: 0},
    )(xt, table, pe_tail, out_sc)
    return out


def kernel(x, table):
    return _impl(x, table)


# speedup vs baseline: 6.5032x; 1.9437x over previous
import math

import numpy as np
import jax
import jax.numpy as jnp
from jax import lax
from jax.experimental import pallas as pl
from jax.experimental.pallas import tpu as pltpu
from jax.experimental.pallas import tpu_sc as plsc

D_DIM = 256
D_HALF = 128
BATCH = 128
SEQ = 1500
SEQ_PAD = 1536
CH = 16
N_STEP = 93
SC_SEQ = CH * N_STEP
TC_TAIL = SEQ - SC_SEQ
N_PAIR = 16
B_PER_P = BATCH // N_PAIR
SCALE = math.sqrt(D_DIM)
LANES = 16
N_G = D_HALF // LANES
B_TC = 8


def _pe_np():
    position = np.arange(0.0, SEQ, dtype=np.float64)[:, None]
    div_term = np.exp(
        np.arange(0.0, D_DIM, 2, dtype=np.float64) * -(math.log(10000.0) / D_DIM)
    )
    ang = position * div_term
    pe = np.zeros((SEQ_PAD, D_DIM), dtype=np.float32)
    pe[:SEQ, 0::2] = np.sin(ang)
    pe[:SEQ, 1::2] = np.cos(ang)
    return pe


_PE = _pe_np()
_PE_T_CONST = np.ascontiguousarray(_PE[:SC_SEQ].reshape(N_STEP, CH, D_DIM))
_PE_TAIL_CONST = np.ascontiguousarray(_PE[SC_SEQ:SEQ])


def _take16(vec, idx):
    dnums = lax.GatherDimensionNumbers(
        offset_dims=(), collapsed_slice_dims=(0,), start_index_map=(0,)
    )
    return lax.gather(
        vec, idx[:, None], dnums, (1,),
        mode=lax.GatherScatterMode.PROMISE_IN_BOUNDS,
    )


def _sc_body(
    x_hbm, table_hbm, pet_hbm, out_hbm,
    idx_v, table_v, wb0, wb1, pe0, pe1,
    wsem0, wsem1, psem0, psem1,
):
    pair = lax.axis_index("s")
    half = lax.axis_index("c")
    b0 = pair * B_PER_P
    d0 = half * D_HALF
    wb = [wb0, wb1]
    peb = [pe0, pe1]
    wsem = [wsem0, wsem1]
    psem = [psem0, psem1]

    pltpu.sync_copy(x_hbm.at[pair], idx_v)
    pltpu.sync_copy(table_hbm.at[:, pl.ds(d0, D_HALF)], table_v)

    lane = lax.iota(jnp.int32, LANES)
    colv = [lane + 16 * g for g in range(N_G)]

    def prefetch_pe(i, p):
        pltpu.async_copy(pet_hbm.at[i, :, pl.ds(d0, D_HALF)], peb[p], psem[p])

    def wait_pe(p):
        pltpu.make_async_copy(
            pet_hbm.at[0, :, pl.ds(0, D_HALF)], peb[p], psem[p]
        ).wait()

    def wait_write(p):
        for b in range(B_PER_P):
            pltpu.make_async_copy(
                wb[p].at[b], out_hbm.at[0, pl.ds(0, CH), pl.ds(0, D_HALF)], wsem[p]
            ).wait()

    def step(i, p, q):
        @pl.when(i >= 2)
        def _():
            wait_write(p)

        @pl.when(i + 1 < N_STEP)
        def _():
            prefetch_pe(i + 1, q)

        wait_pe(p)

        @plsc.parallel_loop(0, CH * B_PER_P, 1)
        def _(sb):
            s = sb // B_PER_P
            b = sb % B_PER_P
            sv = jnp.full((LANES,), s, dtype=jnp.int32)
            iv = idx_v[b, pl.ds(i * CH, CH)]
            spl = _take16(iv, sv)
            for g in range(N_G):
                gv = plsc.load_gather(table_v, [spl, colv[g]])
                pe_vec = peb[p][s, pl.ds(16 * g, LANES)]
                wb[p][b, s, pl.ds(16 * g, LANES)] = gv * SCALE + pe_vec

        for b in range(B_PER_P):
            pltpu.async_copy(
                wb[p].at[b],
                out_hbm.at[b0 + b, pl.ds(i * CH, CH), pl.ds(d0, D_HALF)],
                wsem[p],
            )

    prefetch_pe(0, 0)
    step(0, 0, 1)

    def loop_body(it, carry):
        base = 1 + it * 2
        step(base, 1, 0)
        step(base + 1, 0, 1)
        return carry

    lax.fori_loop(0, (N_STEP - 1) // 2, loop_body, 0)

    wait_write(1)
    wait_write(0)


def _tc_tail_body(xt_ref, table_ref, pe_ref, out_in_ref, out_ref, acc_ref, sem):
    del out_in_ref
    i = pl.program_id(0)
    table = table_ref[...]
    pe = pe_ref[...]
    rows = xt_ref[0, 0, :]
    oh = (
        rows[:, None] == lax.broadcasted_iota(jnp.int32, (B_TC * TC_TAIL, D_DIM), 1)
    )
    acc = jax.lax.dot(
        oh.astype(jnp.float32), table, precision=lax.Precision.HIGHEST
    )
    acc = acc * SCALE + jnp.tile(pe, (B_TC, 1))
    acc_ref[...] = acc.reshape(B_TC, TC_TAIL, D_DIM)
    copy = pltpu.make_async_copy(
        acc_ref,
        out_ref.at[pl.ds(i * B_TC, B_TC), pl.ds(SC_SEQ, TC_TAIL)],
        sem,
    )
    copy.start()
    copy.wait()


@jax.jit
def _impl(x, table):
    pet = jnp.asarray(_PE_T_CONST)
    pe_tail = jnp.asarray(_PE_TAIL_CONST)
    mesh = plsc.VectorSubcoreMesh(core_axis_name="c", subcore_axis_name="s")
    k = pl.kernel(
        _sc_body,
        mesh=mesh,
        out_type=jax.ShapeDtypeStruct((BATCH, SEQ, D_DIM), jnp.float32),
        scratch_types=[
            pltpu.VMEM((B_PER_P, SEQ_PAD), jnp.int32),
            pltpu.VMEM((D_DIM, D_HALF), jnp.float32),
            pltpu.VMEM((B_PER_P, CH, D_HALF), jnp.float32),
            pltpu.VMEM((B_PER_P, CH, D_HALF), jnp.float32),
            pltpu.VMEM((CH, D_HALF), jnp.float32),
            pltpu.VMEM((CH, D_HALF), jnp.float32),
            pltpu.SemaphoreType.DMA,
            pltpu.SemaphoreType.DMA,
            pltpu.SemaphoreType.DMA,
            pltpu.SemaphoreType.DMA,
        ],
        compiler_params=pltpu.CompilerParams(needs_layout_passes=False),
    )
    xp = jnp.pad(x, ((0, 0), (0, SEQ_PAD - SEQ))).reshape(N_PAIR, B_PER_P, SEQ_PAD)
    out_sc = k(xp, table, pet)

    xt = lax.slice(x, (0, SC_SEQ), (BATCH, SEQ)).reshape(
        BATCH // B_TC, 1, B_TC * TC_TAIL
    )
    out = pl.pallas_call(
        _tc_tail_body,
        grid=(BATCH // B_TC,),
        in_specs=[
            pl.BlockSpec((1, 1, B_TC * TC_TAIL), lambda i: (i, 0, 0)),
            pl.BlockSpec((D_DIM, D_DIM), lambda i: (0, 0)),
            pl.BlockSpec((TC_TAIL, D_DIM), lambda i: (0, 0)),
            pl.BlockSpec(memory_space=pl.ANY),
        ],
        out_specs=pl.BlockSpec(memory_space=pl.ANY),
        out_shape=jax.ShapeDtypeStruct((BATCH, SEQ, D_DIM), jnp.float32),
        scratch_shapes=[
            pltpu.VMEM((B_TC, TC_TAIL, D_DIM), jnp.float32),
            pltpu.SemaphoreType.DMA,
        ],
        input_output_aliases={3: 0},
    )(xt, table, pe_tail, out_sc)
    return out


def kernel(x, table):
    return _impl(x, table)


# speedup vs baseline: 6.7777x; 1.0422x over previous
import math

import numpy as np
import jax
import jax.numpy as jnp
from jax import lax
from jax.experimental import pallas as pl
from jax.experimental.pallas import tpu as pltpu
from jax.experimental.pallas import tpu_sc as plsc

D_DIM = 256
D_HALF = 128
BATCH = 128
SEQ = 1500
SEQ_PAD = 1536
CH = 16
N_STEP = 93
SC_SEQ = CH * N_STEP
TC_TAIL = SEQ - SC_SEQ
N_PAIR = 16
B_PER_P = BATCH // N_PAIR
SCALE = math.sqrt(D_DIM)
LANES = 16
N_G = D_HALF // LANES
B_TC = 8


def _pe_np():
    position = np.arange(0.0, SEQ, dtype=np.float64)[:, None]
    div_term = np.exp(
        np.arange(0.0, D_DIM, 2, dtype=np.float64) * -(math.log(10000.0) / D_DIM)
    )
    ang = position * div_term
    pe = np.zeros((SEQ_PAD, D_DIM), dtype=np.float32)
    pe[:SEQ, 0::2] = np.sin(ang)
    pe[:SEQ, 1::2] = np.cos(ang)
    return pe


_PE = _pe_np()
_PE_T_CONST = np.ascontiguousarray(_PE[:SC_SEQ].reshape(N_STEP, CH, D_DIM))
_PE_TAIL_CONST = np.ascontiguousarray(_PE[SC_SEQ:SEQ])


def _take16(vec, idx):
    dnums = lax.GatherDimensionNumbers(
        offset_dims=(), collapsed_slice_dims=(0,), start_index_map=(0,)
    )
    return lax.gather(
        vec, idx[:, None], dnums, (1,),
        mode=lax.GatherScatterMode.PROMISE_IN_BOUNDS,
    )


def _sc_body(
    x_hbm, table_hbm, pet_hbm, out_hbm,
    idx_v, table_v, wb0, wb1, pe0, pe1,
    wsem0, wsem1, psem0, psem1,
):
    pair = lax.axis_index("s")
    half = lax.axis_index("c")
    b0 = pair * B_PER_P
    d0 = half * D_HALF
    wb = [wb0, wb1]
    peb = [pe0, pe1]
    wsem = [wsem0, wsem1]
    psem = [psem0, psem1]

    pltpu.sync_copy(x_hbm.at[pair], idx_v)
    pltpu.sync_copy(table_hbm.at[:, pl.ds(d0, D_HALF)], table_v)

    lane = lax.iota(jnp.int32, LANES)
    colv = [lane + 16 * g for g in range(N_G)]

    def prefetch_pe(i, p):
        pltpu.async_copy(pet_hbm.at[i, :, pl.ds(d0, D_HALF)], peb[p], psem[p])

    def wait_pe(p):
        pltpu.make_async_copy(
            pet_hbm.at[0, :, pl.ds(0, D_HALF)], peb[p], psem[p]
        ).wait()

    def wait_write(p):
        for b in range(B_PER_P):
            pltpu.make_async_copy(
                wb[p].at[b], out_hbm.at[0, pl.ds(0, CH), pl.ds(0, D_HALF)], wsem[p]
            ).wait()

    def step(i, p, q):
        @pl.when(i >= 2)
        def _():
            wait_write(p)

        @pl.when(i + 1 < N_STEP)
        def _():
            prefetch_pe(i + 1, q)

        wait_pe(p)

        @plsc.parallel_loop(0, CH * B_PER_P, 1)
        def _(sb):
            s = sb // B_PER_P
            b = sb % B_PER_P
            sv = jnp.full((LANES,), s, dtype=jnp.int32)
            iv = idx_v[b, pl.ds(i * CH, CH)]
            spl = _take16(iv, sv)
            for g in range(N_G):
                gv = plsc.load_gather(table_v, [spl, colv[g]])
                pe_vec = peb[p][s, pl.ds(16 * g, LANES)]
                wb[p][b, s, pl.ds(16 * g, LANES)] = gv * SCALE + pe_vec

        for b in range(B_PER_P):
            pltpu.async_copy(
                wb[p].at[b],
                out_hbm.at[b0 + b, pl.ds(i * CH, CH), pl.ds(d0, D_HALF)],
                wsem[p],
            )

    prefetch_pe(0, 0)
    step(0, 0, 1)

    def loop_body(it, carry):
        base = 1 + it * 2
        step(base, 1, 0)
        step(base + 1, 0, 1)
        return carry

    lax.fori_loop(0, (N_STEP - 1) // 2, loop_body, 0)

    wait_write(1)
    wait_write(0)


def _tc_tail_body(xt_ref, table_ref, pe_ref, out_in_ref, out_ref, acc_ref, sem):
    del out_in_ref
    table = table_ref[...]
    pe = pe_ref[...]
    rows = xt_ref[0, :]
    n = BATCH * TC_TAIL
    oh = rows[:, None] == lax.broadcasted_iota(jnp.int32, (n, D_DIM), 1)
    acc = jax.lax.dot(
        oh.astype(jnp.float32), table, precision=lax.Precision.HIGHEST
    )
    acc = acc * SCALE + jnp.tile(pe, (BATCH, 1))
    acc_ref[...] = acc.reshape(BATCH, TC_TAIL, D_DIM)
    copy = pltpu.make_async_copy(
        acc_ref,
        out_ref.at[:, pl.ds(SC_SEQ, TC_TAIL)],
        sem,
    )
    copy.start()
    copy.wait()


@jax.jit
def _impl(x, table):
    pet = jnp.asarray(_PE_T_CONST)
    pe_tail = jnp.asarray(_PE_TAIL_CONST)
    mesh = plsc.VectorSubcoreMesh(core_axis_name="c", subcore_axis_name="s")
    k = pl.kernel(
        _sc_body,
        mesh=mesh,
        out_type=jax.ShapeDtypeStruct((BATCH, SEQ, D_DIM), jnp.float32),
        scratch_types=[
            pltpu.VMEM((B_PER_P, SEQ_PAD), jnp.int32),
            pltpu.VMEM((D_DIM, D_HALF), jnp.float32),
            pltpu.VMEM((B_PER_P, CH, D_HALF), jnp.float32),
            pltpu.VMEM((B_PER_P, CH, D_HALF), jnp.float32),
            pltpu.VMEM((CH, D_HALF), jnp.float32),
            pltpu.VMEM((CH, D_HALF), jnp.float32),
            pltpu.SemaphoreType.DMA,
            pltpu.SemaphoreType.DMA,
            pltpu.SemaphoreType.DMA,
            pltpu.SemaphoreType.DMA,
        ],
        compiler_params=pltpu.CompilerParams(needs_layout_passes=False),
    )
    xp = jnp.pad(x, ((0, 0), (0, SEQ_PAD - SEQ))).reshape(N_PAIR, B_PER_P, SEQ_PAD)
    out_sc = k(xp, table, pet)

    xt = lax.slice(x, (0, SC_SEQ), (BATCH, SEQ)).reshape(1, BATCH * TC_TAIL)
    out = pl.pallas_call(
        _tc_tail_body,
        in_specs=[
            pl.BlockSpec((1, BATCH * TC_TAIL), lambda: (0, 0)),
            pl.BlockSpec((D_DIM, D_DIM), lambda: (0, 0)),
            pl.BlockSpec((TC_TAIL, D_DIM), lambda: (0, 0)),
            pl.BlockSpec(memory_space=pl.ANY),
        ],
        out_specs=pl.BlockSpec(memory_space=pl.ANY),
        out_shape=jax.ShapeDtypeStruct((BATCH, SEQ, D_DIM), jnp.float32),
        scratch_shapes=[
            pltpu.VMEM((BATCH, TC_TAIL, D_DIM), jnp.float32),
            pltpu.SemaphoreType.DMA,
        ],
        input_output_aliases={3: 0},
    )(xt, table, pe_tail, out_sc)
    return out


def kernel(x, table):
    return _impl(x, table)


# speedup vs baseline: 6.7813x; 1.0005x over previous
import math

import numpy as np
import jax
import jax.numpy as jnp
from jax import lax
from jax.experimental import pallas as pl
from jax.experimental.pallas import tpu as pltpu
from jax.experimental.pallas import tpu_sc as plsc

D_DIM = 256
D_HALF = 128
BATCH = 128
SEQ = 1500
SEQ_PAD = 1536
CH = 16
N_STEP = 93
SC_SEQ = CH * N_STEP
TC_TAIL = SEQ - SC_SEQ
N_PAIR = 16
B_PER_P = BATCH // N_PAIR
SCALE = math.sqrt(D_DIM)
LANES = 16
N_G = D_HALF // LANES


def _pe_np():
    position = np.arange(0.0, SEQ, dtype=np.float64)[:, None]
    div_term = np.exp(
        np.arange(0.0, D_DIM, 2, dtype=np.float64) * -(math.log(10000.0) / D_DIM)
    )
    ang = position * div_term
    pe = np.zeros((SEQ_PAD, D_DIM), dtype=np.float32)
    pe[:SEQ, 0::2] = np.sin(ang)
    pe[:SEQ, 1::2] = np.cos(ang)
    return pe


_PE = _pe_np()
_PE_T_CONST = np.ascontiguousarray(_PE[:SC_SEQ].reshape(N_STEP, CH, D_DIM))
_PE_TAIL_CONST = np.ascontiguousarray(_PE[SC_SEQ:SEQ])


def _take16(vec, idx):
    dnums = lax.GatherDimensionNumbers(
        offset_dims=(), collapsed_slice_dims=(0,), start_index_map=(0,)
    )
    return lax.gather(
        vec, idx[:, None], dnums, (1,),
        mode=lax.GatherScatterMode.PROMISE_IN_BOUNDS,
    )


def _sc_body(
    x_hbm, table_hbm, pet_hbm, out_hbm,
    idx_v, table_v, wb0, wb1, pe0, pe1,
    wsem0, wsem1, psem0, psem1,
):
    pair = lax.axis_index("s")
    half = lax.axis_index("c")
    b0 = pair * B_PER_P
    d0 = half * D_HALF
    wb = [wb0, wb1]
    peb = [pe0, pe1]
    wsem = [wsem0, wsem1]
    psem = [psem0, psem1]

    pltpu.sync_copy(x_hbm.at[pair], idx_v)
    pltpu.sync_copy(table_hbm.at[:, pl.ds(d0, D_HALF)], table_v)

    lane = lax.iota(jnp.int32, LANES)
    colv = [lane + 16 * g for g in range(N_G)]

    def prefetch_pe(i, p):
        pltpu.async_copy(pet_hbm.at[i, :, pl.ds(d0, D_HALF)], peb[p], psem[p])

    def wait_pe(p):
        pltpu.make_async_copy(
            pet_hbm.at[0, :, pl.ds(0, D_HALF)], peb[p], psem[p]
        ).wait()

    def wait_write(p):
        for b in range(B_PER_P):
            pltpu.make_async_copy(
                wb[p].at[b], out_hbm.at[0, pl.ds(0, CH), pl.ds(0, D_HALF)], wsem[p]
            ).wait()

    def step(i, p, q):
        @pl.when(i >= 2)
        def _():
            wait_write(p)

        @pl.when(i + 1 < N_STEP)
        def _():
            prefetch_pe(i + 1, q)

        wait_pe(p)

        @plsc.parallel_loop(0, CH * B_PER_P, 1)
        def _(sb):
            s = sb // B_PER_P
            b = sb % B_PER_P
            sv = jnp.full((LANES,), s, dtype=jnp.int32)
            iv = idx_v[b, pl.ds(i * CH, CH)]
            spl = _take16(iv, sv)
            for g in range(N_G):
                gv = plsc.load_gather(table_v, [spl, colv[g]])
                pe_vec = peb[p][s, pl.ds(16 * g, LANES)]
                wb[p][b, s, pl.ds(16 * g, LANES)] = gv * SCALE + pe_vec

        for b in range(B_PER_P):
            pltpu.async_copy(
                wb[p].at[b],
                out_hbm.at[b0 + b, pl.ds(i * CH, CH), pl.ds(d0, D_HALF)],
                wsem[p],
            )

    prefetch_pe(0, 0)
    step(0, 0, 1)

    def loop_body(it, carry):
        base = 1 + it * 2
        step(base, 1, 0)
        step(base + 1, 0, 1)
        return carry

    lax.fori_loop(0, (N_STEP - 1) // 2, loop_body, 0)

    wait_write(1)
    wait_write(0)


def _tc_tail_body(xt_ref, table_ref, pe_ref, out_in_ref, out_ref, acc_ref, sem):
    del out_in_ref
    table = table_ref[...]
    pe = pe_ref[...]
    rows = xt_ref[0, :]
    n = BATCH * TC_TAIL
    oh = rows[:, None] == lax.broadcasted_iota(jnp.int32, (n, D_DIM), 1)
    acc = jax.lax.dot(
        oh.astype(jnp.float32), table, precision=lax.Precision.HIGHEST
    )
    acc = acc * SCALE + jnp.tile(pe, (BATCH, 1))
    acc_ref[...] = acc.reshape(BATCH, TC_TAIL, D_DIM)
    copy = pltpu.make_async_copy(
        acc_ref,
        out_ref.at[:, pl.ds(SC_SEQ, TC_TAIL)],
        sem,
    )
    copy.start()
    copy.wait()


@jax.jit
def _impl(x, table):
    pet = jnp.asarray(_PE_T_CONST)
    pe_tail = jnp.asarray(_PE_TAIL_CONST)
    mesh = plsc.VectorSubcoreMesh(core_axis_name="c", subcore_axis_name="s")
    k = pl.kernel(
        _sc_body,
        mesh=mesh,
        out_type=jax.ShapeDtypeStruct((BATCH, SEQ, D_DIM), jnp.float32),
        scratch_types=[
            pltpu.VMEM((B_PER_P, SEQ_PAD), jnp.int32),
            pltpu.VMEM((D_DIM, D_HALF), jnp.float32),
            pltpu.VMEM((B_PER_P, CH, D_HALF), jnp.float32),
            pltpu.VMEM((B_PER_P, CH, D_HALF), jnp.float32),
            pltpu.VMEM((CH, D_HALF), jnp.float32),
            pltpu.VMEM((CH, D_HALF), jnp.float32),
            pltpu.SemaphoreType.DMA,
            pltpu.SemaphoreType.DMA,
            pltpu.SemaphoreType.DMA,
            pltpu.SemaphoreType.DMA,
        ],
        compiler_params=pltpu.CompilerParams(needs_layout_passes=False),
    )
    xp = jnp.pad(x, ((0, 0), (0, SEQ_PAD - SEQ))).reshape(N_PAIR, B_PER_P, SEQ_PAD)
    out_sc = k(xp, table, pet)

    xt = lax.slice(x, (0, SC_SEQ), (BATCH, SEQ)).reshape(1, BATCH * TC_TAIL)
    out = pl.pallas_call(
        _tc_tail_body,
        in_specs=[
            pl.BlockSpec((1, BATCH * TC_TAIL), lambda: (0, 0)),
            pl.BlockSpec((D_DIM, D_DIM), lambda: (0, 0)),
            pl.BlockSpec((TC_TAIL, D_DIM), lambda: (0, 0)),
            pl.BlockSpec(memory_space=pl.ANY),
        ],
        out_specs=pl.BlockSpec(memory_space=pl.ANY),
        out_shape=jax.ShapeDtypeStruct((BATCH, SEQ, D_DIM), jnp.float32),
        scratch_shapes=[
            pltpu.VMEM((BATCH, TC_TAIL, D_DIM), jnp.float32),
            pltpu.SemaphoreType.DMA,
        ],
        input_output_aliases={3: 0},
    )(xt, table, pe_tail, out_sc)
    return out


def kernel(x, table):
    return _impl(x, table)
